# baseline (speedup 1.0000x reference)
import functools

import jax
import jax.numpy as jnp
from jax import lax
from jax.experimental import pallas as pl
from jax.experimental.pallas import tpu as pltpu
from jax.experimental.pallas import tpu_sc as plsc

_B = 16384
_NC = 2
_NS = 16
_NW = _NC * _NS
_COLS = _B // _NW
_STEPS = _COLS // 16

_mesh = plsc.VectorSubcoreMesh(
    core_axis_name="c", subcore_axis_name="s", num_cores=_NC, num_subcores=_NS
)


@functools.partial(
    pl.kernel,
    out_type=jax.ShapeDtypeStruct((4, _B), jnp.float32),
    mesh=_mesh,
    compiler_params=pltpu.CompilerParams(
        needs_layout_passes=False, use_tc_tiling_on_sc=True),
    scratch_types=[
        pltpu.VMEM((5, _COLS), jnp.int32),
        pltpu.VMEM((2, _COLS), jnp.float32),
        pltpu.VMEM((2, _COLS), jnp.float32),
        pltpu.VMEM((2, 31), jnp.float32),
        pltpu.VMEM((2, 31), jnp.float32),
        pltpu.VMEM((2, 31), jnp.float32),
        pltpu.VMEM((2, 31), jnp.float32),
        pltpu.VMEM((4, _COLS), jnp.float32),
    ],
)
def _rank_sc(stim_hbm, gw1_hbm, gw0_hbm, e0_hbm, e1_hbm, e2_hbm, e3_hbm,
             out_hbm, stim_v, gw1_v, gw0_v, e0_v, e1_v, e2_v, e3_v, out_v):
    wid = lax.axis_index("s") * _NC + lax.axis_index("c")
    base = wid * _COLS

    pltpu.sync_copy(stim_hbm.at[:, pl.ds(base, _COLS)], stim_v)
    pltpu.sync_copy(gw1_hbm.at[:, pl.ds(base, _COLS)], gw1_v)
    pltpu.sync_copy(gw0_hbm.at[:, pl.ds(base, _COLS)], gw0_v)
    pltpu.sync_copy(e0_hbm, e0_v)
    pltpu.sync_copy(e1_hbm, e1_v)
    pltpu.sync_copy(e2_hbm, e2_v)
    pltpu.sync_copy(e3_hbm, e3_v)

    zero = jnp.zeros((16,), jnp.int32)
    one = jnp.full((16,), 1, jnp.int32)

    @plsc.parallel_loop(0, _STEPS, unroll=4)
    def step(i):
        rbase = i * 16
        g0 = gw1_v[0, pl.ds(rbase, 16)]
        a0 = gw0_v[0, pl.ds(rbase, 16)]
        a1 = 1.0 - a0
        g1 = 1.0 - g0
        c0 = a0 * g0
        c1 = a0 * g1
        c2 = a1 * g0
        c3 = a1 * g1

        zx = []
        zy = []
        for j in range(5):
            s = stim_v[j, pl.ds(rbase, 16)]
            vx = (c0 * plsc.load_gather(e0_v, [zero, s])
                  + c1 * plsc.load_gather(e1_v, [zero, s])
                  + c2 * plsc.load_gather(e2_v, [zero, s])
                  + c3 * plsc.load_gather(e3_v, [zero, s]))
            vy = (c0 * plsc.load_gather(e0_v, [one, s])
                  + c1 * plsc.load_gather(e1_v, [one, s])
                  + c2 * plsc.load_gather(e2_v, [one, s])
                  + c3 * plsc.load_gather(e3_v, [one, s]))
            zx.append(vx)
            zy.append(vy)

        es = []
        for j in range(1, 5):
            dx = zx[0] - zx[j]
            dy = zy[0] - zy[j]
            q = 1.2 * dx * dx + 0.8 * dy * dy
            bits = lax.bitcast_convert_type(q, jnp.int32)
            bits = 0x5F3759DF - (bits >> 1)
            r = lax.bitcast_convert_type(bits, jnp.float32)
            hq = 0.5 * q
            for _ in range(3):
                r = r * (1.5 - hq * r * r)
            dist = q * r
            es.append(jnp.exp(-10.0 * dist))

        inv = 1.0 / (es[0] + es[1] + es[2] + es[3])
        for j in range(4):
            out_v[j, pl.ds(rbase, 16)] = es[j] * inv

    pltpu.sync_copy(out_v, out_hbm.at[:, pl.ds(base, _COLS)])


def kernel(given4rank1_stimulus_set, percept_gate_weights_1,
           percept_gate_weights_0, E0, E1, E2, E3):
    stim_t = given4rank1_stimulus_set.astype(jnp.int32).T
    out = _rank_sc(stim_t, percept_gate_weights_1.T, percept_gate_weights_0.T,
                   E0.T, E1.T, E2.T, E3.T)
    return out.T


# speedup vs baseline: 1.0946x; 1.0946x over previous
import functools

import jax
import jax.numpy as jnp
from jax import lax
from jax.experimental import pallas as pl
from jax.experimental.pallas import tpu as pltpu
from jax.experimental.pallas import tpu_sc as plsc

_B = 16384
_NC = 2
_NS = 16
_NW = _NC * _NS
_COLS = _B // _NW
_STEPS = _COLS // 16

_mesh = plsc.VectorSubcoreMesh(
    core_axis_name="c", subcore_axis_name="s", num_cores=_NC, num_subcores=_NS
)


@functools.partial(
    pl.kernel,
    out_type=jax.ShapeDtypeStruct((4, _B), jnp.float32),
    mesh=_mesh,
    compiler_params=pltpu.CompilerParams(
        needs_layout_passes=False, use_tc_tiling_on_sc=True),
    scratch_types=[
        pltpu.VMEM((5, _COLS), jnp.int32),
        pltpu.VMEM((2, _COLS), jnp.float32),
        pltpu.VMEM((2, _COLS), jnp.float32),
        pltpu.VMEM((2, 31), jnp.float32),
        pltpu.VMEM((2, 31), jnp.float32),
        pltpu.VMEM((2, 31), jnp.float32),
        pltpu.VMEM((2, 31), jnp.float32),
        pltpu.VMEM((4, _COLS), jnp.float32),
    ],
)
def _rank_sc(stim_hbm, gw1_hbm, gw0_hbm, e0_hbm, e1_hbm, e2_hbm, e3_hbm,
             out_hbm, stim_v, gw1_v, gw0_v, e0_v, e1_v, e2_v, e3_v, out_v):
    wid = lax.axis_index("s") * _NC + lax.axis_index("c")
    base = wid * _COLS

    pltpu.sync_copy(stim_hbm.at[:, pl.ds(base, _COLS)], stim_v)
    pltpu.sync_copy(gw1_hbm.at[:, pl.ds(base, _COLS)], gw1_v)
    pltpu.sync_copy(gw0_hbm.at[:, pl.ds(base, _COLS)], gw0_v)
    pltpu.sync_copy(e0_hbm, e0_v)
    pltpu.sync_copy(e1_hbm, e1_v)
    pltpu.sync_copy(e2_hbm, e2_v)
    pltpu.sync_copy(e3_hbm, e3_v)

    zero = jnp.zeros((16,), jnp.int32)
    one = jnp.full((16,), 1, jnp.int32)

    @plsc.parallel_loop(0, _STEPS, unroll=4)
    def step(i):
        rbase = i * 16
        g0 = gw1_v[0, pl.ds(rbase, 16)]
        a0 = gw0_v[0, pl.ds(rbase, 16)]
        a1 = 1.0 - a0
        g1 = 1.0 - g0
        c0 = a0 * g0
        c1 = a0 * g1
        c2 = a1 * g0
        c3 = a1 * g1

        zx = []
        zy = []
        for j in range(5):
            s = stim_v[j, pl.ds(rbase, 16)]
            vx = (c0 * plsc.load_gather(e0_v, [zero, s])
                  + c1 * plsc.load_gather(e1_v, [zero, s])
                  + c2 * plsc.load_gather(e2_v, [zero, s])
                  + c3 * plsc.load_gather(e3_v, [zero, s]))
            vy = (c0 * plsc.load_gather(e0_v, [one, s])
                  + c1 * plsc.load_gather(e1_v, [one, s])
                  + c2 * plsc.load_gather(e2_v, [one, s])
                  + c3 * plsc.load_gather(e3_v, [one, s]))
            zx.append(vx)
            zy.append(vy)

        es = []
        for j in range(1, 5):
            dx = zx[0] - zx[j]
            dy = zy[0] - zy[j]
            q = 1.2 * dx * dx + 0.8 * dy * dy
            q = jnp.maximum(q, jnp.float32(1e-30))
            bits = lax.bitcast_convert_type(q, jnp.int32)
            bits = 0x5F3759DF - (bits >> 1)
            r = lax.bitcast_convert_type(bits, jnp.float32)
            hq = 0.5 * q
            for _ in range(3):
                r = r * (1.5 - hq * r * r)
            dist = q * r
            es.append(jnp.exp(-10.0 * dist))

        inv = 1.0 / (es[0] + es[1] + es[2] + es[3])
        for j in range(4):
            out_v[j, pl.ds(rbase, 16)] = es[j] * inv

    pltpu.sync_copy(out_v, out_hbm.at[:, pl.ds(base, _COLS)])


def kernel(given4rank1_stimulus_set, percept_gate_weights_1,
           percept_gate_weights_0, E0, E1, E2, E3):
    stim_t = given4rank1_stimulus_set.astype(jnp.int32).T
    out = _rank_sc(stim_t, percept_gate_weights_1.T, percept_gate_weights_0.T,
                   E0.T, E1.T, E2.T, E3.T)
    return out.T


# speedup vs baseline: 1.2264x; 1.1204x over previous
import functools

import jax
import jax.numpy as jnp
from jax import lax
from jax.experimental import pallas as pl
from jax.experimental.pallas import tpu as pltpu
from jax.experimental.pallas import tpu_sc as plsc

_B = 16384
_NC = 2
_NS = 16
_NW = _NC * _NS
_COLS = _B // _NW
_STEPS = _COLS // 16

_mesh = plsc.VectorSubcoreMesh(
    core_axis_name="c", subcore_axis_name="s", num_cores=_NC, num_subcores=_NS
)


@functools.partial(
    pl.kernel,
    out_type=jax.ShapeDtypeStruct((4, _B), jnp.float32),
    mesh=_mesh,
    compiler_params=pltpu.CompilerParams(
        needs_layout_passes=False, use_tc_tiling_on_sc=True),
    scratch_types=[
        pltpu.VMEM((5, _COLS), jnp.int32),
        pltpu.VMEM((2, _COLS), jnp.float32),
        pltpu.VMEM((2, _COLS), jnp.float32),
        pltpu.VMEM((2, 31), jnp.float32),
        pltpu.VMEM((2, 31), jnp.float32),
        pltpu.VMEM((2, 31), jnp.float32),
        pltpu.VMEM((2, 31), jnp.float32),
        pltpu.VMEM((4, _COLS), jnp.float32),
        pltpu.SemaphoreType.DMA,
    ],
)
def _rank_sc(stim_hbm, gw1_hbm, gw0_hbm, e0_hbm, e1_hbm, e2_hbm, e3_hbm,
             out_hbm, stim_v, gw1_v, gw0_v, e0_v, e1_v, e2_v, e3_v, out_v,
             sem):
    wid = lax.axis_index("s") * _NC + lax.axis_index("c")
    base = wid * _COLS

    copies = [
        pltpu.async_copy(stim_hbm.at[:, pl.ds(base, _COLS)], stim_v, sem),
        pltpu.async_copy(gw1_hbm.at[:, pl.ds(base, _COLS)], gw1_v, sem),
        pltpu.async_copy(gw0_hbm.at[:, pl.ds(base, _COLS)], gw0_v, sem),
        pltpu.async_copy(e0_hbm, e0_v, sem),
        pltpu.async_copy(e1_hbm, e1_v, sem),
        pltpu.async_copy(e2_hbm, e2_v, sem),
        pltpu.async_copy(e3_hbm, e3_v, sem),
    ]
    for c in copies:
        c.wait()

    zero = jnp.zeros((16,), jnp.int32)
    one = jnp.full((16,), 1, jnp.int32)

    @plsc.parallel_loop(0, _STEPS, unroll=4)
    def step(i):
        rbase = i * 16
        g0 = gw1_v[0, pl.ds(rbase, 16)]
        a0 = gw0_v[0, pl.ds(rbase, 16)]
        a1 = 1.0 - a0
        g1 = 1.0 - g0
        c0 = a0 * g0
        c1 = a0 * g1
        c2 = a1 * g0
        c3 = a1 * g1

        zx = []
        zy = []
        for j in range(5):
            s = stim_v[j, pl.ds(rbase, 16)]
            vx = (c0 * plsc.load_gather(e0_v, [zero, s])
                  + c1 * plsc.load_gather(e1_v, [zero, s])
                  + c2 * plsc.load_gather(e2_v, [zero, s])
                  + c3 * plsc.load_gather(e3_v, [zero, s]))
            vy = (c0 * plsc.load_gather(e0_v, [one, s])
                  + c1 * plsc.load_gather(e1_v, [one, s])
                  + c2 * plsc.load_gather(e2_v, [one, s])
                  + c3 * plsc.load_gather(e3_v, [one, s]))
            zx.append(vx)
            zy.append(vy)

        es = []
        for j in range(1, 5):
            dx = zx[0] - zx[j]
            dy = zy[0] - zy[j]
            q = 1.2 * dx * dx + 0.8 * dy * dy
            q = jnp.maximum(q, jnp.float32(1e-30))
            bits = lax.bitcast_convert_type(q, jnp.int32)
            bits = 0x5F3759DF - (bits >> 1)
            r = lax.bitcast_convert_type(bits, jnp.float32)
            hq = 0.5 * q
            for _ in range(3):
                r = r * (1.5 - hq * r * r)
            dist = q * r
            es.append(jnp.exp(-10.0 * dist))

        inv = 1.0 / (es[0] + es[1] + es[2] + es[3])
        for j in range(4):
            out_v[j, pl.ds(rbase, 16)] = es[j] * inv

    pltpu.sync_copy(out_v, out_hbm.at[:, pl.ds(base, _COLS)])


def kernel(given4rank1_stimulus_set, percept_gate_weights_1,
           percept_gate_weights_0, E0, E1, E2, E3):
    stim_t = given4rank1_stimulus_set.astype(jnp.int32).T
    out = _rank_sc(stim_t, percept_gate_weights_1.T, percept_gate_weights_0.T,
                   E0.T, E1.T, E2.T, E3.T)
    return out.T
